# baseline (speedup 1.0000x reference)
import functools

import jax
import jax.numpy as jnp
from jax import lax
from jax.experimental import pallas as pl
from jax.experimental.pallas import tpu as pltpu
from jax.experimental.pallas import tpu_sc as plsc

N_USERS = 25000
N_ITEMS = 25000
NN = N_USERS + N_ITEMS
D = 64
HD = D // 2
E = 800000

NC = 2
NS = 16
EPT = E // NS
CH = 400
NCHUNK = EPT // CH
G = 5
SUB = CH // G
ACC_ROWS = 50176
ZSTR = ACC_ROWS // NS
WSTR = 3120
WTAIL = NN - NS * WSTR


def _zero_rows(rows):
    z = jnp.zeros((16,), jnp.float32)

    @plsc.parallel_loop(0, CH, unroll=8)
    def _(i):
        for j in range(HD // 16):
            rows[i, pl.ds(j * 16, 16)] = z


def _gathers(x_hbm, srcf, rows, sem):
    return [
        pltpu.async_copy(x_hbm.at[srcf.at[pl.ds(g * SUB, SUB)]],
                         rows.at[pl.ds(g * SUB, SUB)], sem)
        for g in range(G)
    ]


def _scatters(acc, dst2, rows, sem):
    return [
        pltpu.async_copy(rows.at[pl.ds(g * SUB, SUB)],
                         acc.at[dst2.at[g]], sem, add=True)
        for g in range(G)
    ]


def _sw_loads(src_hbm, w_hbm, off, srcf, wf, sem):
    return [pltpu.async_copy(src_hbm.at[pl.ds(off, CH)], srcf, sem),
            pltpu.async_copy(w_hbm.at[pl.ds(off, CH)], wf, sem)]


def _wait_sw_loads(src_hbm, w_hbm, srcf, wf, sem):
    pltpu.make_async_copy(src_hbm.at[pl.ds(0, CH)], srcf, sem).wait()
    pltpu.make_async_copy(w_hbm.at[pl.ds(0, CH)], wf, sem).wait()


def _dst_loads(dst_hbm, off, dst2, sem):
    return [
        pltpu.async_copy(dst_hbm.at[pl.ds(off + g * SUB, SUB)], dst2.at[g], sem)
        for g in range(G)
    ]


def _wait_dst_loads(dst_hbm, dst2, sem):
    for g in range(G):
        pltpu.make_async_copy(dst_hbm.at[pl.ds(0, SUB)], dst2.at[g],
                              sem).wait()


def _cond(pred, fn):
    if isinstance(pred, bool):
        if pred:
            fn()
    else:
        pl.when(pred)(fn)


def _edge_pass(x_hbm, dst_hbm, src_hbm, w_hbm, acc,
               dst2, srcf, wf, rows, isems, dsems, gsems, ssems, tbase, s):
    ebase = s * EPT

    def adjust_src(b):
        @plsc.parallel_loop(0, CH // 16, unroll=5)
        def _(i):
            sl = pl.ds(i * 16, 16)
            srcf[b][sl] = srcf[b][sl] + tbase

    def wait_scatters(b):
        for g in range(G):
            pltpu.make_async_copy(rows.at[b].at[pl.ds(g * SUB, SUB)],
                                  acc.at[dst2[b].at[g]], ssems[b]).wait()

    def do_chunk(k, b, stage, waitprev, more):
        nb = 1 - b
        rows_b = rows.at[b]

        for g in range(G):
            pltpu.make_async_copy(x_hbm.at[srcf[b].at[pl.ds(g * SUB, SUB)]],
                                  rows_b.at[pl.ds(g * SUB, SUB)],
                                  gsems[b]).wait()

        if stage:
            _cond(waitprev, lambda: wait_scatters(nb))
            _wait_sw_loads(src_hbm, w_hbm, srcf[nb], wf[nb], isems[nb])
            _dst_loads(dst_hbm, ebase + (k + 1) * CH, dst2[nb], dsems[nb])
            adjust_src(nb)
            _gathers(x_hbm, srcf[nb], rows.at[nb], gsems[nb])

        @plsc.parallel_loop(0, CH, unroll=16)
        def _(e):
            wsp = plsc.load_gather(wf[b], [jnp.broadcast_to(e, (16,))])
            for j in range(HD // 16):
                sl = pl.ds(j * 16, 16)
                rows_b[e, sl] = rows_b[e, sl] * wsp

        _cond(more, lambda: [
            None for _ in _sw_loads(src_hbm, w_hbm, ebase + (k + 2) * CH,
                                    srcf[b], wf[b], isems[b])])

        _wait_dst_loads(dst_hbm, dst2[b], dsems[b])
        _scatters(acc, dst2[b], rows_b, ssems[b])

    for cp in _sw_loads(src_hbm, w_hbm, ebase, srcf[0], wf[0], isems[0]):
        cp.wait()
    adjust_src(0)
    _dst_loads(dst_hbm, ebase, dst2[0], dsems[0])
    _gathers(x_hbm, srcf[0], rows.at[0], gsems[0])
    _sw_loads(src_hbm, w_hbm, ebase + CH, srcf[1], wf[1], isems[1])

    do_chunk(0, 0, stage=True, waitprev=False, more=True)

    @pl.loop(1, NCHUNK - 2, step=2)
    def _(k):
        do_chunk(k, 1, stage=True, waitprev=True, more=True)
        do_chunk(k + 1, 0, stage=True, waitprev=True, more=True)

    do_chunk(NCHUNK - 2, 1, stage=True, waitprev=True, more=False)
    do_chunk(NCHUNK - 1, 0, stage=False, waitprev=True, more=False)

    wait_scatters(1)
    wait_scatters(0)


def _zero_acc(acc, rows, s):
    _zero_rows(rows.at[0])
    for o in range(0, 2800, CH):
        pltpu.sync_copy(rows.at[0], acc.at[pl.ds(s * ZSTR + o, CH)])
    pltpu.sync_copy(rows.at[0].at[pl.ds(0, ZSTR - 2800)],
                    acc.at[pl.ds(s * ZSTR + 2800, ZSTR - 2800)])


_WCHUNKS = tuple((o, CH) for o in range(0, 2800, CH)) + ((2800, WSTR - 2800),)


def _fused_body(x_hbm, dst_hbm, src_hbm, w_hbm, e1_hbm, e2_hbm, mean_hbm,
                acc, dst2a, dst2b, srcfa, srcfb, wfa, wfb, rows,
                isem0, isem1, dsem0, dsem1, gsem0, gsem1, ssem0, ssem1):
    c = lax.axis_index("c")
    s = lax.axis_index("s")
    tbase = c * NN

    sems = ((isem0, isem1), (dsem0, dsem1), (gsem0, gsem1), (ssem0, ssem1))
    bufs = ((dst2a, dst2b), (srcfa, srcfb), (wfa, wfb), rows)

    _zero_acc(acc, rows, s)
    plsc.subcore_barrier()
    _edge_pass(x_hbm, dst_hbm, src_hbm, w_hbm, acc, *bufs, *sems, tbase, s)
    plsc.subcore_barrier()

    def writeback(o, sz):
        pltpu.sync_copy(acc.at[pl.ds(s * WSTR + o, sz)],
                        rows.at[0].at[pl.ds(0, sz)])
        pltpu.sync_copy(rows.at[0].at[pl.ds(0, sz)],
                        e1_hbm.at[pl.ds(tbase + s * WSTR + o, sz)])

    for o, sz in _WCHUNKS:
        writeback(o, sz)

    @pl.when(s == NS - 1)
    def _():
        pltpu.sync_copy(acc.at[pl.ds(NS * WSTR, WTAIL)],
                        rows.at[0].at[pl.ds(0, WTAIL)])
        pltpu.sync_copy(rows.at[0].at[pl.ds(0, WTAIL)],
                        e1_hbm.at[pl.ds(tbase + NS * WSTR, WTAIL)])

    plsc.subcore_barrier()
    _zero_acc(acc, rows, s)
    plsc.subcore_barrier()

    _edge_pass(e1_hbm, dst_hbm, src_hbm, w_hbm, acc, *bufs, *sems, tbase, s)
    plsc.subcore_barrier()

    third = jnp.float32(1.0 / 3.0)

    def combine(o, sz):
        lrow = s * WSTR + o
        grow = tbase + lrow
        r0 = rows.at[0]
        r1 = rows.at[1]
        pltpu.sync_copy(acc.at[pl.ds(lrow, sz)], r0.at[pl.ds(0, sz)])
        pltpu.sync_copy(r0.at[pl.ds(0, sz)], e2_hbm.at[pl.ds(grow, sz)])
        pltpu.sync_copy(x_hbm.at[pl.ds(grow, sz)], r1.at[pl.ds(0, sz)])

        @plsc.parallel_loop(0, sz, unroll=8)
        def _(r):
            for j in range(HD // 16):
                sl = pl.ds(j * 16, 16)
                r1[r, sl] = r1[r, sl] + r0[r, sl]

        pltpu.sync_copy(e1_hbm.at[pl.ds(grow, sz)], r0.at[pl.ds(0, sz)])

        @plsc.parallel_loop(0, sz, unroll=8)
        def _(r):
            for j in range(HD // 16):
                sl = pl.ds(j * 16, 16)
                r1[r, sl] = (r1[r, sl] + r0[r, sl]) * third

        pltpu.sync_copy(r1.at[pl.ds(0, sz)], mean_hbm.at[pl.ds(grow, sz)])

    for o, sz in _WCHUNKS:
        combine(o, sz)

    @pl.when(s == NS - 1)
    def _():
        combine(WSTR, WTAIL)


_MESH = plsc.VectorSubcoreMesh(core_axis_name="c", subcore_axis_name="s")

_SCRATCH = (
    pltpu.VMEM_SHARED((ACC_ROWS, HD), jnp.float32),
    pltpu.VMEM((G, SUB), jnp.int32),
    pltpu.VMEM((G, SUB), jnp.int32),
    pltpu.VMEM((CH,), jnp.int32),
    pltpu.VMEM((CH,), jnp.int32),
    pltpu.VMEM((CH,), jnp.float32),
    pltpu.VMEM((CH,), jnp.float32),
    pltpu.VMEM((2, CH, HD), jnp.float32),
    pltpu.SemaphoreType.DMA,
    pltpu.SemaphoreType.DMA,
    pltpu.SemaphoreType.DMA,
    pltpu.SemaphoreType.DMA,
    pltpu.SemaphoreType.DMA,
    pltpu.SemaphoreType.DMA,
    pltpu.SemaphoreType.DMA,
    pltpu.SemaphoreType.DMA,
)

_PARAMS = pltpu.CompilerParams(needs_layout_passes=False,
                               use_tc_tiling_on_sc=False)


@functools.partial(
    pl.kernel,
    out_type=(
        jax.ShapeDtypeStruct((NC * NN, HD), jnp.float32),
        jax.ShapeDtypeStruct((NC * NN, HD), jnp.float32),
        jax.ShapeDtypeStruct((NC * NN, HD), jnp.float32),
    ),
    mesh=_MESH,
    scratch_types=list(_SCRATCH),
    compiler_params=_PARAMS,
)
def _fused(x_hbm, dst_hbm, src_hbm, w_hbm, e1_hbm, e2_hbm, mean_hbm,
           *scratch):
    _fused_body(x_hbm, dst_hbm, src_hbm, w_hbm, e1_hbm, e2_hbm, mean_hbm,
                *scratch)


def kernel(users_emb, items_emb, edge_index, edge_weight):
    x0 = jnp.concatenate([users_emb, items_emb], axis=0)
    xs = jnp.concatenate([x0[:, :HD], x0[:, HD:]], axis=0)
    dst = edge_index[0]
    src = edge_index[1]
    _, e2s, means = _fused(xs, dst, src, edge_weight)
    light = jnp.concatenate([means[:NN], means[NN:]], axis=1)
    return (light[:N_USERS], light[N_USERS:], e2s[:NN], e2s[NN:])


# speedup vs baseline: 1.0814x; 1.0814x over previous
import functools

import jax
import jax.numpy as jnp
from jax import lax
from jax.experimental import pallas as pl
from jax.experimental.pallas import tpu as pltpu
from jax.experimental.pallas import tpu_sc as plsc

N_USERS = 25000
N_ITEMS = 25000
NN = N_USERS + N_ITEMS
D = 64
HD = D // 2
E = 800000

NC = 2
NS = 16
EPT = E // NS
CH = 400
NCHUNK = EPT // CH
G = 5
SUB = CH // G
ACC_ROWS = 50176
ZSTR = ACC_ROWS // NS
WSTR = 3120
WTAIL = NN - NS * WSTR


def _zero_rows(rows):
    z = jnp.zeros((16,), jnp.float32)

    @plsc.parallel_loop(0, CH, unroll=8)
    def _(i):
        for j in range(HD // 16):
            rows[i, pl.ds(j * 16, 16)] = z


def _gathers(x_hbm, srcf, rows, sem):
    return [
        pltpu.async_copy(x_hbm.at[srcf.at[pl.ds(g * SUB, SUB)]],
                         rows.at[pl.ds(g * SUB, SUB)], sem)
        for g in range(G)
    ]


def _scatters(acc, dst2, rows, sem):
    return [
        pltpu.async_copy(rows.at[pl.ds(g * SUB, SUB)],
                         acc.at[dst2.at[g]], sem, add=True)
        for g in range(G)
    ]


def _sw_loads(src_hbm, w_hbm, off, srcf, wf, sem):
    return [pltpu.async_copy(src_hbm.at[pl.ds(off, CH)], srcf, sem),
            pltpu.async_copy(w_hbm.at[pl.ds(off, CH)], wf, sem)]


def _wait_sw_loads(src_hbm, w_hbm, srcf, wf, sem):
    pltpu.make_async_copy(src_hbm.at[pl.ds(0, CH)], srcf, sem).wait()
    pltpu.make_async_copy(w_hbm.at[pl.ds(0, CH)], wf, sem).wait()


def _dst_loads(dst_hbm, off, dst2, sem):
    return [
        pltpu.async_copy(dst_hbm.at[pl.ds(off + g * SUB, SUB)], dst2.at[g], sem)
        for g in range(G)
    ]


def _wait_dst_loads(dst_hbm, dst2, sem):
    for g in range(G):
        pltpu.make_async_copy(dst_hbm.at[pl.ds(0, SUB)], dst2.at[g],
                              sem).wait()


def _cond(pred, fn):
    if isinstance(pred, bool):
        if pred:
            fn()
    else:
        pl.when(pred)(fn)


def _edge_pass(x_hbm, dst_hbm, src_hbm, w_hbm, acc,
               dst2, srcf, wf, rows, isems, dsems, gsems, ssems, tbase, s):
    ebase = s * EPT

    def adjust_src(b):
        @plsc.parallel_loop(0, CH // 16, unroll=5)
        def _(i):
            sl = pl.ds(i * 16, 16)
            srcf[b][sl] = srcf[b][sl] + tbase

    def wait_scatters(b):
        for g in range(G):
            pltpu.make_async_copy(rows.at[b].at[pl.ds(g * SUB, SUB)],
                                  acc.at[dst2[b].at[g]], ssems[b]).wait()

    def do_chunk(k, b, stage, waitprev, more):
        nb = 1 - b
        rows_b = rows.at[b]

        for g in range(G):
            pltpu.make_async_copy(x_hbm.at[srcf[b].at[pl.ds(g * SUB, SUB)]],
                                  rows_b.at[pl.ds(g * SUB, SUB)],
                                  gsems[b]).wait()

        if stage:
            _cond(waitprev, lambda: wait_scatters(nb))
            _wait_sw_loads(src_hbm, w_hbm, srcf[nb], wf[nb], isems[nb])
            _dst_loads(dst_hbm, ebase + (k + 1) * CH, dst2[nb], dsems[nb])
            adjust_src(nb)
            _gathers(x_hbm, srcf[nb], rows.at[nb], gsems[nb])

        @plsc.parallel_loop(0, CH, unroll=16)
        def _(e):
            wsp = plsc.load_gather(wf[b], [jnp.broadcast_to(e, (16,))])
            for j in range(HD // 16):
                sl = pl.ds(j * 16, 16)
                rows_b[e, sl] = rows_b[e, sl] * wsp

        _cond(more, lambda: [
            None for _ in _sw_loads(src_hbm, w_hbm, ebase + (k + 2) * CH,
                                    srcf[b], wf[b], isems[b])])

        _wait_dst_loads(dst_hbm, dst2[b], dsems[b])
        _scatters(acc, dst2[b], rows_b, ssems[b])

    for cp in _sw_loads(src_hbm, w_hbm, ebase, srcf[0], wf[0], isems[0]):
        cp.wait()
    adjust_src(0)
    _dst_loads(dst_hbm, ebase, dst2[0], dsems[0])
    _gathers(x_hbm, srcf[0], rows.at[0], gsems[0])
    _sw_loads(src_hbm, w_hbm, ebase + CH, srcf[1], wf[1], isems[1])

    do_chunk(0, 0, stage=True, waitprev=False, more=True)

    @pl.loop(1, NCHUNK - 2, step=2)
    def _(k):
        do_chunk(k, 1, stage=True, waitprev=True, more=True)
        do_chunk(k + 1, 0, stage=True, waitprev=True, more=True)

    do_chunk(NCHUNK - 2, 1, stage=True, waitprev=True, more=False)
    do_chunk(NCHUNK - 1, 0, stage=False, waitprev=True, more=False)

    wait_scatters(1)
    wait_scatters(0)


def _zero_acc(acc, rows, s):
    _zero_rows(rows.at[0])
    for o in range(0, 2800, CH):
        pltpu.sync_copy(rows.at[0], acc.at[pl.ds(s * ZSTR + o, CH)])
    pltpu.sync_copy(rows.at[0].at[pl.ds(0, ZSTR - 2800)],
                    acc.at[pl.ds(s * ZSTR + 2800, ZSTR - 2800)])


_WCHUNKS = tuple((o, CH) for o in range(0, 2800, CH)) + ((2800, WSTR - 2800),)


def _fused_body(x_hbm, dst_hbm, src_hbm, w_hbm, e1_hbm, e2_hbm, mean_hbm,
                acc, dst2a, dst2b, srcfa, srcfb, wfa, wfb, rows,
                isem0, isem1, dsem0, dsem1, gsem0, gsem1, ssem0, ssem1):
    c = lax.axis_index("c")
    s = lax.axis_index("s")
    tbase = c * NN
    tbase_col = c * HD

    sems = ((isem0, isem1), (dsem0, dsem1), (gsem0, gsem1), (ssem0, ssem1))
    bufs = ((dst2a, dst2b), (srcfa, srcfb), (wfa, wfb), rows)

    _zero_acc(acc, rows, s)
    plsc.subcore_barrier()
    _edge_pass(x_hbm, dst_hbm, src_hbm, w_hbm, acc, *bufs, *sems, tbase, s)
    plsc.subcore_barrier()

    def writeback(o, sz):
        pltpu.sync_copy(acc.at[pl.ds(s * WSTR + o, sz)],
                        rows.at[0].at[pl.ds(0, sz)])
        pltpu.sync_copy(rows.at[0].at[pl.ds(0, sz)],
                        e1_hbm.at[pl.ds(tbase + s * WSTR + o, sz)])

    for o, sz in _WCHUNKS:
        writeback(o, sz)

    @pl.when(s == NS - 1)
    def _():
        pltpu.sync_copy(acc.at[pl.ds(NS * WSTR, WTAIL)],
                        rows.at[0].at[pl.ds(0, WTAIL)])
        pltpu.sync_copy(rows.at[0].at[pl.ds(0, WTAIL)],
                        e1_hbm.at[pl.ds(tbase + NS * WSTR, WTAIL)])

    plsc.subcore_barrier()
    _zero_acc(acc, rows, s)
    plsc.subcore_barrier()

    _edge_pass(e1_hbm, dst_hbm, src_hbm, w_hbm, acc, *bufs, *sems, tbase, s)
    plsc.subcore_barrier()

    third = jnp.float32(1.0 / 3.0)

    def combine(o, sz):
        lrow = s * WSTR + o
        grow = tbase + lrow
        r0 = rows.at[0]
        r1 = rows.at[1]
        pltpu.sync_copy(acc.at[pl.ds(lrow, sz)], r0.at[pl.ds(0, sz)])
        pltpu.sync_copy(r0.at[pl.ds(0, sz)], e2_hbm.at[pl.ds(grow, sz)])
        pltpu.sync_copy(x_hbm.at[pl.ds(grow, sz)], r1.at[pl.ds(0, sz)])

        @plsc.parallel_loop(0, sz, unroll=8)
        def _(r):
            for j in range(HD // 16):
                sl = pl.ds(j * 16, 16)
                r1[r, sl] = r1[r, sl] + r0[r, sl]

        pltpu.sync_copy(e1_hbm.at[pl.ds(grow, sz)], r0.at[pl.ds(0, sz)])

        @plsc.parallel_loop(0, sz, unroll=8)
        def _(r):
            for j in range(HD // 16):
                sl = pl.ds(j * 16, 16)
                r1[r, sl] = (r1[r, sl] + r0[r, sl]) * third

        pltpu.sync_copy(r1.at[pl.ds(0, sz)],
                        mean_hbm.at[pl.ds(lrow, sz), pl.ds(tbase_col, HD)])

    for o, sz in _WCHUNKS:
        combine(o, sz)

    @pl.when(s == NS - 1)
    def _():
        combine(WSTR, WTAIL)


_MESH = plsc.VectorSubcoreMesh(core_axis_name="c", subcore_axis_name="s")

_SCRATCH = (
    pltpu.VMEM_SHARED((ACC_ROWS, HD), jnp.float32),
    pltpu.VMEM((G, SUB), jnp.int32),
    pltpu.VMEM((G, SUB), jnp.int32),
    pltpu.VMEM((CH,), jnp.int32),
    pltpu.VMEM((CH,), jnp.int32),
    pltpu.VMEM((CH,), jnp.float32),
    pltpu.VMEM((CH,), jnp.float32),
    pltpu.VMEM((2, CH, HD), jnp.float32),
    pltpu.SemaphoreType.DMA,
    pltpu.SemaphoreType.DMA,
    pltpu.SemaphoreType.DMA,
    pltpu.SemaphoreType.DMA,
    pltpu.SemaphoreType.DMA,
    pltpu.SemaphoreType.DMA,
    pltpu.SemaphoreType.DMA,
    pltpu.SemaphoreType.DMA,
)

_PARAMS = pltpu.CompilerParams(needs_layout_passes=False,
                               use_tc_tiling_on_sc=False)


@functools.partial(
    pl.kernel,
    out_type=(
        jax.ShapeDtypeStruct((NC * NN, HD), jnp.float32),
        jax.ShapeDtypeStruct((NC * NN, HD), jnp.float32),
        jax.ShapeDtypeStruct((NN, D), jnp.float32),
    ),
    mesh=_MESH,
    scratch_types=list(_SCRATCH),
    compiler_params=_PARAMS,
)
def _fused(x_hbm, dst_hbm, src_hbm, w_hbm, e1_hbm, e2_hbm, mean_hbm,
           *scratch):
    _fused_body(x_hbm, dst_hbm, src_hbm, w_hbm, e1_hbm, e2_hbm, mean_hbm,
                *scratch)


def kernel(users_emb, items_emb, edge_index, edge_weight):
    x0 = jnp.concatenate([users_emb, items_emb], axis=0)
    xs = jnp.concatenate([x0[:, :HD], x0[:, HD:]], axis=0)
    dst = edge_index[0]
    src = edge_index[1]
    _, e2s, light = _fused(xs, dst, src, edge_weight)
    return (light[:N_USERS], light[N_USERS:], e2s[:NN], e2s[NN:])


# speedup vs baseline: 1.1295x; 1.0445x over previous
import functools

import jax
import jax.numpy as jnp
from jax import lax
from jax.experimental import pallas as pl
from jax.experimental.pallas import tpu as pltpu
from jax.experimental.pallas import tpu_sc as plsc

N_USERS = 25000
N_ITEMS = 25000
NN = N_USERS + N_ITEMS
D = 64
HD = D // 2
E = 800000

NC = 2
NS = 16
EPT = E // NS
CH = 400
NCHUNK = EPT // CH
G = 5
SUB = CH // G
ACC_ROWS = 50176
ZSTR = ACC_ROWS // NS
WSTR = 3120
WTAIL = NN - NS * WSTR


def _zero_rows(rows):
    z = jnp.zeros((16,), jnp.float32)

    @plsc.parallel_loop(0, CH, unroll=8)
    def _(i):
        for j in range(HD // 16):
            rows[i, pl.ds(j * 16, 16)] = z


def _gathers(x_hbm, srcf, rows, sem):
    return [
        pltpu.async_copy(x_hbm.at[srcf.at[pl.ds(g * SUB, SUB)]],
                         rows.at[pl.ds(g * SUB, SUB)], sem)
        for g in range(G)
    ]


def _scatters(acc, dst2, rows, sem):
    return [
        pltpu.async_copy(rows.at[pl.ds(g * SUB, SUB)],
                         acc.at[dst2.at[g]], sem, add=True)
        for g in range(G)
    ]


def _sw_loads(src_hbm, w_hbm, off, srcf, wf, sem):
    return [pltpu.async_copy(src_hbm.at[pl.ds(off, CH)], srcf, sem),
            pltpu.async_copy(w_hbm.at[pl.ds(off, CH)], wf, sem)]


def _wait_sw_loads(src_hbm, w_hbm, srcf, wf, sem):
    pltpu.make_async_copy(src_hbm.at[pl.ds(0, CH)], srcf, sem).wait()
    pltpu.make_async_copy(w_hbm.at[pl.ds(0, CH)], wf, sem).wait()


def _dst_loads(dst_hbm, off, dst2, sem):
    return [
        pltpu.async_copy(dst_hbm.at[pl.ds(off + g * SUB, SUB)], dst2.at[g], sem)
        for g in range(G)
    ]


def _wait_dst_loads(dst_hbm, dst2, sem):
    for g in range(G):
        pltpu.make_async_copy(dst_hbm.at[pl.ds(0, SUB)], dst2.at[g],
                              sem).wait()


def _cond(pred, fn):
    if isinstance(pred, bool):
        if pred:
            fn()
    else:
        pl.when(pred)(fn)


def _edge_pass(x_hbm, dst_hbm, src_hbm, w_hbm, acc,
               dst2, srcf, wf, rows, isems, dsems, gsems, ssems, tbase, s):
    ebase = s * EPT

    def adjust_src(b):
        @plsc.parallel_loop(0, CH // 16, unroll=5)
        def _(i):
            sl = pl.ds(i * 16, 16)
            srcf[b][sl] = srcf[b][sl] + tbase

    def wait_scatters(b):
        for g in range(G):
            pltpu.make_async_copy(rows.at[b].at[pl.ds(g * SUB, SUB)],
                                  acc.at[dst2[b].at[g]], ssems[b]).wait()

    def do_chunk(k, b, stage, waitprev, more):
        nb = 1 - b
        rows_b = rows.at[b]

        for g in range(G):
            pltpu.make_async_copy(x_hbm.at[srcf[b].at[pl.ds(g * SUB, SUB)]],
                                  rows_b.at[pl.ds(g * SUB, SUB)],
                                  gsems[b]).wait()

        if stage:
            _cond(waitprev, lambda: wait_scatters(nb))
            _wait_sw_loads(src_hbm, w_hbm, srcf[nb], wf[nb], isems[nb])
            _dst_loads(dst_hbm, ebase + (k + 1) * CH, dst2[nb], dsems[nb])
            adjust_src(nb)
            _gathers(x_hbm, srcf[nb], rows.at[nb], gsems[nb])

        @plsc.parallel_loop(0, CH, unroll=16)
        def _(e):
            wsp = plsc.load_gather(wf[b], [jnp.broadcast_to(e, (16,))])
            for j in range(HD // 16):
                sl = pl.ds(j * 16, 16)
                rows_b[e, sl] = rows_b[e, sl] * wsp

        _cond(more, lambda: [
            None for _ in _sw_loads(src_hbm, w_hbm, ebase + (k + 2) * CH,
                                    srcf[b], wf[b], isems[b])])

        _wait_dst_loads(dst_hbm, dst2[b], dsems[b])
        _scatters(acc, dst2[b], rows_b, ssems[b])

    for cp in _sw_loads(src_hbm, w_hbm, ebase, srcf[0], wf[0], isems[0]):
        cp.wait()
    adjust_src(0)
    _dst_loads(dst_hbm, ebase, dst2[0], dsems[0])
    _gathers(x_hbm, srcf[0], rows.at[0], gsems[0])
    _sw_loads(src_hbm, w_hbm, ebase + CH, srcf[1], wf[1], isems[1])

    do_chunk(0, 0, stage=True, waitprev=False, more=True)

    @pl.loop(1, NCHUNK - 2, step=2)
    def _(k):
        do_chunk(k, 1, stage=True, waitprev=True, more=True)
        do_chunk(k + 1, 0, stage=True, waitprev=True, more=True)

    do_chunk(NCHUNK - 2, 1, stage=True, waitprev=True, more=False)
    do_chunk(NCHUNK - 1, 0, stage=False, waitprev=True, more=False)

    wait_scatters(1)
    wait_scatters(0)


def _zero_acc(acc, rows, s):
    _zero_rows(rows.at[0])
    for o in range(0, 2800, CH):
        pltpu.sync_copy(rows.at[0], acc.at[pl.ds(s * ZSTR + o, CH)])
    pltpu.sync_copy(rows.at[0].at[pl.ds(0, ZSTR - 2800)],
                    acc.at[pl.ds(s * ZSTR + 2800, ZSTR - 2800)])


_WCHUNKS = tuple((o, CH) for o in range(0, 2800, CH)) + ((2800, WSTR - 2800),)


def _fused_body(x_hbm, dst_hbm, src_hbm, w_hbm, e1_hbm, e2_hbm,
                users_hbm, items_hbm,
                acc, dst2a, dst2b, srcfa, srcfb, wfa, wfb, rows,
                isem0, isem1, dsem0, dsem1, gsem0, gsem1, ssem0, ssem1):
    c = lax.axis_index("c")
    s = lax.axis_index("s")
    tbase = c * NN
    tbase_col = c * HD

    sems = ((isem0, isem1), (dsem0, dsem1), (gsem0, gsem1), (ssem0, ssem1))
    bufs = ((dst2a, dst2b), (srcfa, srcfb), (wfa, wfb), rows)

    _zero_acc(acc, rows, s)
    plsc.subcore_barrier()
    _edge_pass(x_hbm, dst_hbm, src_hbm, w_hbm, acc, *bufs, *sems, tbase, s)
    plsc.subcore_barrier()

    def writeback(o, sz):
        pltpu.sync_copy(acc.at[pl.ds(s * WSTR + o, sz)],
                        rows.at[0].at[pl.ds(0, sz)])
        pltpu.sync_copy(rows.at[0].at[pl.ds(0, sz)],
                        e1_hbm.at[pl.ds(tbase + s * WSTR + o, sz)])

    for o, sz in _WCHUNKS:
        writeback(o, sz)

    @pl.when(s == NS - 1)
    def _():
        pltpu.sync_copy(acc.at[pl.ds(NS * WSTR, WTAIL)],
                        rows.at[0].at[pl.ds(0, WTAIL)])
        pltpu.sync_copy(rows.at[0].at[pl.ds(0, WTAIL)],
                        e1_hbm.at[pl.ds(tbase + NS * WSTR, WTAIL)])

    plsc.subcore_barrier()
    _zero_acc(acc, rows, s)
    plsc.subcore_barrier()

    _edge_pass(e1_hbm, dst_hbm, src_hbm, w_hbm, acc, *bufs, *sems, tbase, s)
    plsc.subcore_barrier()

    third = jnp.float32(1.0 / 3.0)

    def combine(o, sz):
        lrow = s * WSTR + o
        grow = tbase + lrow
        r0 = rows.at[0]
        r1 = rows.at[1]
        pltpu.sync_copy(acc.at[pl.ds(lrow, sz)], r0.at[pl.ds(0, sz)])
        pltpu.sync_copy(r0.at[pl.ds(0, sz)], e2_hbm.at[pl.ds(grow, sz)])
        pltpu.sync_copy(x_hbm.at[pl.ds(grow, sz)], r1.at[pl.ds(0, sz)])

        @plsc.parallel_loop(0, sz, unroll=8)
        def _(r):
            for j in range(HD // 16):
                sl = pl.ds(j * 16, 16)
                r1[r, sl] = r1[r, sl] + r0[r, sl]

        pltpu.sync_copy(e1_hbm.at[pl.ds(grow, sz)], r0.at[pl.ds(0, sz)])

        @plsc.parallel_loop(0, sz, unroll=8)
        def _(r):
            for j in range(HD // 16):
                sl = pl.ds(j * 16, 16)
                r1[r, sl] = (r1[r, sl] + r0[r, sl]) * third

        lo_users = lrow + sz <= N_USERS
        hi_items = lrow >= N_USERS

        @pl.when(lo_users)
        def _():
            pltpu.sync_copy(
                r1.at[pl.ds(0, sz)],
                users_hbm.at[pl.ds(lrow, sz), pl.ds(tbase_col, HD)])

        @pl.when(hi_items)
        def _():
            pltpu.sync_copy(
                r1.at[pl.ds(0, sz)],
                items_hbm.at[pl.ds(lrow - N_USERS, sz), pl.ds(tbase_col, HD)])

        if sz > 40:
            @pl.when(jnp.logical_not(lo_users | hi_items))
            def _():
                pltpu.sync_copy(
                    r1.at[pl.ds(0, 40)],
                    users_hbm.at[pl.ds(lrow, 40), pl.ds(tbase_col, HD)])
                pltpu.sync_copy(
                    r1.at[pl.ds(40, sz - 40)],
                    items_hbm.at[pl.ds(lrow + 40 - N_USERS, sz - 40),
                                 pl.ds(tbase_col, HD)])

    for o, sz in _WCHUNKS:
        combine(o, sz)

    @pl.when(s == NS - 1)
    def _():
        combine(WSTR, WTAIL)


_MESH = plsc.VectorSubcoreMesh(core_axis_name="c", subcore_axis_name="s")

_SCRATCH = (
    pltpu.VMEM_SHARED((ACC_ROWS, HD), jnp.float32),
    pltpu.VMEM((G, SUB), jnp.int32),
    pltpu.VMEM((G, SUB), jnp.int32),
    pltpu.VMEM((CH,), jnp.int32),
    pltpu.VMEM((CH,), jnp.int32),
    pltpu.VMEM((CH,), jnp.float32),
    pltpu.VMEM((CH,), jnp.float32),
    pltpu.VMEM((2, CH, HD), jnp.float32),
    pltpu.SemaphoreType.DMA,
    pltpu.SemaphoreType.DMA,
    pltpu.SemaphoreType.DMA,
    pltpu.SemaphoreType.DMA,
    pltpu.SemaphoreType.DMA,
    pltpu.SemaphoreType.DMA,
    pltpu.SemaphoreType.DMA,
    pltpu.SemaphoreType.DMA,
)

_PARAMS = pltpu.CompilerParams(needs_layout_passes=False,
                               use_tc_tiling_on_sc=False)


@functools.partial(
    pl.kernel,
    out_type=(
        jax.ShapeDtypeStruct((NC * NN, HD), jnp.float32),
        jax.ShapeDtypeStruct((NC * NN, HD), jnp.float32),
        jax.ShapeDtypeStruct((N_USERS, D), jnp.float32),
        jax.ShapeDtypeStruct((N_ITEMS, D), jnp.float32),
    ),
    mesh=_MESH,
    scratch_types=list(_SCRATCH),
    compiler_params=_PARAMS,
)
def _fused(x_hbm, dst_hbm, src_hbm, w_hbm, e1_hbm, e2_hbm,
           users_hbm, items_hbm, *scratch):
    _fused_body(x_hbm, dst_hbm, src_hbm, w_hbm, e1_hbm, e2_hbm,
                users_hbm, items_hbm, *scratch)


def kernel(users_emb, items_emb, edge_index, edge_weight):
    x0 = jnp.concatenate([users_emb, items_emb], axis=0)
    xs = jnp.concatenate([x0[:, :HD], x0[:, HD:]], axis=0)
    dst = edge_index[0]
    src = edge_index[1]
    _, e2s, users, items = _fused(xs, dst, src, edge_weight)
    return (users, items, e2s[:NN], e2s[NN:])
